# baseline (speedup 1.0000x reference)
import functools

import jax
import jax.numpy as jnp
from jax import lax
from jax.experimental import pallas as pl
from jax.experimental.pallas import tpu as pltpu
from jax.experimental.pallas import tpu_sc as plsc

BATCH = 1024
EMBED_DIM = 32
VOCAB = 100000
CLIP_MAX_NORM = 10.0

_NC = 2
_NS = 16
_NW = _NC * _NS
_BPW = BATCH // _NW


@functools.lru_cache(maxsize=None)
def _make_sc_gather():
    mesh = plsc.VectorSubcoreMesh(core_axis_name="c", subcore_axis_name="s")

    @functools.partial(
        pl.kernel,
        mesh=mesh,
        out_type=jax.ShapeDtypeStruct((BATCH, EMBED_DIM), jnp.float32),
        scratch_types=[
            pltpu.VMEM((_BPW,), jnp.int32),
            pltpu.VMEM((_BPW, EMBED_DIM), jnp.float32),
            pltpu.SemaphoreType.DMA,
        ],
        compiler_params=pltpu.CompilerParams(use_tc_tiling_on_sc=False),
    )
    def gather(table_hbm, idx_hbm, out_hbm, idx_v, rows_v, sem):
        wid = lax.axis_index("s") * _NC + lax.axis_index("c")
        base = wid * _BPW
        pltpu.sync_copy(idx_hbm.at[pl.ds(base, _BPW)], idx_v)
        pltpu.async_copy(table_hbm.at[idx_v], rows_v, sem).wait()
        pltpu.sync_copy(rows_v, out_hbm.at[pl.ds(base, _BPW)])

    return gather


def _decoder_body(e_ref, w_ref, b_ref, logits_ref, h_ref):
    e = e_ref[...]
    norm = jnp.sqrt(jnp.sum(e * e, axis=1, keepdims=True))
    coef = jnp.minimum(CLIP_MAX_NORM / (norm + 1e-06), 1.0)
    h = e * coef

    @pl.when(pl.program_id(0) == 0)
    def _():
        h_ref[...] = h

    logits_ref[...] = lax.dot_general(
        h, w_ref[...], (((1,), (1,)), ((), ())),
        preferred_element_type=jnp.float32,
    ) + b_ref[...]


@functools.lru_cache(maxsize=None)
def _make_decoder(nb: int):
    grid = (pl.cdiv(VOCAB, nb),)
    return pl.pallas_call(
        _decoder_body,
        grid=grid,
        in_specs=[
            pl.BlockSpec((BATCH, EMBED_DIM), lambda n: (0, 0)),
            pl.BlockSpec((nb, EMBED_DIM), lambda n: (n, 0)),
            pl.BlockSpec((1, nb), lambda n: (0, n)),
        ],
        out_specs=[
            pl.BlockSpec((BATCH, nb), lambda n: (0, n)),
            pl.BlockSpec((BATCH, EMBED_DIM), lambda n: (0, 0)),
        ],
        out_shape=[
            jax.ShapeDtypeStruct((BATCH, VOCAB), jnp.float32),
            jax.ShapeDtypeStruct((BATCH, EMBED_DIM), jnp.float32),
        ],
        compiler_params=pltpu.CompilerParams(
            dimension_semantics=("parallel",),
        ),
    )


def kernel(x, emb_table, W_dec, b_dec):
    e = _make_sc_gather()(emb_table, x.astype(jnp.int32))
    logits, h = _make_decoder(2048)(e, W_dec, b_dec.reshape(1, VOCAB))
    return (logits, h, e)


# speedup vs baseline: 2.9265x; 2.9265x over previous
import functools

import jax
import jax.numpy as jnp
from jax import lax
from jax.experimental import pallas as pl
from jax.experimental.pallas import tpu as pltpu
from jax.experimental.pallas import tpu_sc as plsc

BATCH = 1024
EMBED_DIM = 32
VOCAB = 100000
CLIP_MAX_NORM = 10.0

_NC = 2
_NS = 16
_NW = _NC * _NS
_BPW = BATCH // _NW


@functools.lru_cache(maxsize=None)
def _make_sc_gather():
    mesh = plsc.VectorSubcoreMesh(core_axis_name="c", subcore_axis_name="s")

    @functools.partial(
        pl.kernel,
        mesh=mesh,
        out_type=jax.ShapeDtypeStruct((BATCH, EMBED_DIM), jnp.float32),
        scratch_types=[
            pltpu.VMEM((_BPW,), jnp.int32),
            pltpu.VMEM((_BPW, EMBED_DIM), jnp.float32),
            pltpu.SemaphoreType.DMA,
        ],
        compiler_params=pltpu.CompilerParams(use_tc_tiling_on_sc=False),
    )
    def gather(table_hbm, idx_hbm, out_hbm, idx_v, rows_v, sem):
        wid = lax.axis_index("s") * _NC + lax.axis_index("c")
        base = wid * _BPW
        pltpu.sync_copy(idx_hbm.at[pl.ds(base, _BPW)], idx_v)
        pltpu.async_copy(table_hbm.at[idx_v], rows_v, sem).wait()
        pltpu.sync_copy(rows_v, out_hbm.at[pl.ds(base, _BPW)])

    return gather


def _decoder_body(e_ref, wt_ref, b_ref, out_ref, h_ref):
    e = e_ref[...]
    norm = jnp.sqrt(jnp.sum(e * e, axis=1, keepdims=True))
    coef = jnp.minimum(CLIP_MAX_NORM / (norm + 1e-06), 1.0)
    h = e * coef

    @pl.when(pl.program_id(0) == 0)
    def _():
        h_ref[...] = h

    acc = lax.dot_general(
        wt_ref[...], h, (((0,), (1,)), ((), ())),
        preferred_element_type=jnp.float32,
    )
    out_ref[...] = acc + jnp.transpose(b_ref[...])


@functools.lru_cache(maxsize=None)
def _make_decoder(nb: int):
    grid = (pl.cdiv(VOCAB, nb),)
    return pl.pallas_call(
        _decoder_body,
        grid=grid,
        in_specs=[
            pl.BlockSpec((BATCH, EMBED_DIM), lambda n: (0, 0)),
            pl.BlockSpec((EMBED_DIM, nb), lambda n: (0, n)),
            pl.BlockSpec((1, nb), lambda n: (0, n)),
        ],
        out_specs=[
            pl.BlockSpec((nb, BATCH), lambda n: (n, 0)),
            pl.BlockSpec((BATCH, EMBED_DIM), lambda n: (0, 0)),
        ],
        out_shape=[
            jax.ShapeDtypeStruct((VOCAB, BATCH), jnp.float32),
            jax.ShapeDtypeStruct((BATCH, EMBED_DIM), jnp.float32),
        ],
        compiler_params=pltpu.CompilerParams(
            dimension_semantics=("parallel",),
        ),
    )


def kernel(x, emb_table, W_dec, b_dec):
    e = _make_sc_gather()(emb_table, x.astype(jnp.int32))
    logits_t, h = _make_decoder(2048)(
        e, W_dec.T, b_dec.reshape(1, VOCAB)
    )
    return (logits_t.T, h, e)


# speedup vs baseline: 2.9614x; 1.0119x over previous
import functools

import jax
import jax.numpy as jnp
from jax import lax
from jax.experimental import pallas as pl
from jax.experimental.pallas import tpu as pltpu
from jax.experimental.pallas import tpu_sc as plsc

BATCH = 1024
EMBED_DIM = 32
LANE = 128
VOCAB = 100000
CLIP_MAX_NORM = 10.0

_NC = 2
_NS = 16
_NW = _NC * _NS
_BPW = BATCH // _NW


@functools.lru_cache(maxsize=None)
def _make_sc_gather():
    mesh = plsc.VectorSubcoreMesh(core_axis_name="c", subcore_axis_name="s")

    @functools.partial(
        pl.kernel,
        mesh=mesh,
        out_type=jax.ShapeDtypeStruct((BATCH, LANE), jnp.float32),
        scratch_types=[
            pltpu.VMEM((_BPW,), jnp.int32),
            pltpu.VMEM((_BPW, LANE), jnp.float32),
            pltpu.SemaphoreType.DMA,
        ],
    )
    def gather(table_hbm, idx_hbm, out_hbm, idx_v, rows_v, sem):
        wid = lax.axis_index("s") * _NC + lax.axis_index("c")
        base = wid * _BPW
        pltpu.sync_copy(idx_hbm.at[pl.ds(base, _BPW)], idx_v)
        pltpu.async_copy(table_hbm.at[idx_v], rows_v, sem).wait()
        pltpu.sync_copy(rows_v, out_hbm.at[pl.ds(base, _BPW)])

    return gather


def _decoder_body(e_ref, wt_ref, b_ref, out_ref, h_ref, e_out_ref):
    e = e_ref[:, :EMBED_DIM]
    norm = jnp.sqrt(jnp.sum(e * e, axis=1, keepdims=True))
    coef = jnp.minimum(CLIP_MAX_NORM / (norm + 1e-06), 1.0)
    h = e * coef

    @pl.when(pl.program_id(0) == 0)
    def _():
        h_ref[...] = h
        e_out_ref[...] = e

    acc = lax.dot_general(
        wt_ref[...], h, (((0,), (1,)), ((), ())),
        preferred_element_type=jnp.float32,
    )
    out_ref[...] = acc + jnp.transpose(b_ref[...])


@functools.lru_cache(maxsize=None)
def _make_decoder(nb: int):
    grid = (pl.cdiv(VOCAB, nb),)
    return pl.pallas_call(
        _decoder_body,
        grid=grid,
        in_specs=[
            pl.BlockSpec((BATCH, LANE), lambda n: (0, 0)),
            pl.BlockSpec((EMBED_DIM, nb), lambda n: (0, n)),
            pl.BlockSpec((1, nb), lambda n: (0, n)),
        ],
        out_specs=[
            pl.BlockSpec((nb, BATCH), lambda n: (n, 0)),
            pl.BlockSpec((BATCH, EMBED_DIM), lambda n: (0, 0)),
            pl.BlockSpec((BATCH, EMBED_DIM), lambda n: (0, 0)),
        ],
        out_shape=[
            jax.ShapeDtypeStruct((VOCAB, BATCH), jnp.float32),
            jax.ShapeDtypeStruct((BATCH, EMBED_DIM), jnp.float32),
            jax.ShapeDtypeStruct((BATCH, EMBED_DIM), jnp.float32),
        ],
        compiler_params=pltpu.CompilerParams(
            dimension_semantics=("parallel",),
        ),
    )


def kernel(x, emb_table, W_dec, b_dec):
    emb_pad = jnp.pad(emb_table, ((0, 0), (0, LANE - EMBED_DIM)))
    e128 = _make_sc_gather()(emb_pad, x.astype(jnp.int32))
    logits_t, h, e = _make_decoder(2048)(
        e128, W_dec.T, b_dec.reshape(1, VOCAB)
    )
    return (logits_t.T, h, e)


# speedup vs baseline: 2.9705x; 1.0031x over previous
import functools

import jax
import jax.numpy as jnp
from jax import lax
from jax.experimental import pallas as pl
from jax.experimental.pallas import tpu as pltpu
from jax.experimental.pallas import tpu_sc as plsc

BATCH = 1024
EMBED_DIM = 32
LANE = 128
VOCAB = 100000
CLIP_MAX_NORM = 10.0

_NC = 2
_NS = 16
_NW = _NC * _NS
_BPW = BATCH // _NW


@functools.lru_cache(maxsize=None)
def _make_sc_gather():
    mesh = plsc.VectorSubcoreMesh(core_axis_name="c", subcore_axis_name="s")

    @functools.partial(
        pl.kernel,
        mesh=mesh,
        out_type=jax.ShapeDtypeStruct((BATCH, LANE), jnp.float32),
        scratch_types=[
            pltpu.VMEM((_BPW,), jnp.int32),
            pltpu.VMEM((_BPW, LANE), jnp.float32),
            pltpu.SemaphoreType.DMA,
        ],
    )
    def gather(table_hbm, idx_hbm, out_hbm, idx_v, rows_v, sem):
        wid = lax.axis_index("s") * _NC + lax.axis_index("c")
        base = wid * _BPW
        pltpu.sync_copy(idx_hbm.at[pl.ds(base, _BPW)], idx_v)
        pltpu.async_copy(table_hbm.at[idx_v], rows_v, sem).wait()
        pltpu.sync_copy(rows_v, out_hbm.at[pl.ds(base, _BPW)])

    return gather


def _decoder_body(e_ref, wt_ref, b_ref, out_ref, h_ref, e_out_ref):
    e = e_ref[:, :EMBED_DIM]
    norm = jnp.sqrt(jnp.sum(e * e, axis=1, keepdims=True))
    coef = jnp.minimum(CLIP_MAX_NORM / (norm + 1e-06), 1.0)
    h = e * coef

    @pl.when(pl.program_id(0) == 0)
    def _():
        h_ref[...] = h
        e_out_ref[...] = e

    acc = lax.dot_general(
        wt_ref[...].astype(jnp.bfloat16), h.astype(jnp.bfloat16),
        (((0,), (1,)), ((), ())),
        preferred_element_type=jnp.float32,
    )
    out_ref[...] = acc + jnp.transpose(b_ref[...])


@functools.lru_cache(maxsize=None)
def _make_decoder(nb: int):
    grid = (pl.cdiv(VOCAB, nb),)
    return pl.pallas_call(
        _decoder_body,
        grid=grid,
        in_specs=[
            pl.BlockSpec((BATCH, LANE), lambda n: (0, 0)),
            pl.BlockSpec((EMBED_DIM, nb), lambda n: (0, n)),
            pl.BlockSpec((1, nb), lambda n: (0, n)),
        ],
        out_specs=[
            pl.BlockSpec((nb, BATCH), lambda n: (n, 0)),
            pl.BlockSpec((BATCH, EMBED_DIM), lambda n: (0, 0)),
            pl.BlockSpec((BATCH, EMBED_DIM), lambda n: (0, 0)),
        ],
        out_shape=[
            jax.ShapeDtypeStruct((VOCAB, BATCH), jnp.float32),
            jax.ShapeDtypeStruct((BATCH, EMBED_DIM), jnp.float32),
            jax.ShapeDtypeStruct((BATCH, EMBED_DIM), jnp.float32),
        ],
        compiler_params=pltpu.CompilerParams(
            dimension_semantics=("parallel",),
        ),
    )


def kernel(x, emb_table, W_dec, b_dec):
    emb_pad = jnp.pad(emb_table, ((0, 0), (0, LANE - EMBED_DIM)))
    e128 = _make_sc_gather()(emb_pad, x.astype(jnp.int32))
    logits_t, h, e = _make_decoder(2048)(
        e128, W_dec.T, b_dec.reshape(1, VOCAB)
    )
    return (logits_t.T, h, e)


# speedup vs baseline: 2.9774x; 1.0023x over previous
import functools

import jax
import jax.numpy as jnp
from jax import lax
from jax.experimental import pallas as pl
from jax.experimental.pallas import tpu as pltpu
from jax.experimental.pallas import tpu_sc as plsc

BATCH = 1024
EMBED_DIM = 32
LANE = 128
VOCAB = 100000
CLIP_MAX_NORM = 10.0

_NC = 2
_NS = 16
_NW = _NC * _NS
_BPW = BATCH // _NW


@functools.lru_cache(maxsize=None)
def _make_sc_gather():
    mesh = plsc.VectorSubcoreMesh(core_axis_name="c", subcore_axis_name="s")

    @functools.partial(
        pl.kernel,
        mesh=mesh,
        out_type=jax.ShapeDtypeStruct((BATCH, LANE), jnp.float32),
        scratch_types=[
            pltpu.VMEM((_BPW,), jnp.int32),
            pltpu.VMEM((_BPW, LANE), jnp.float32),
            pltpu.SemaphoreType.DMA,
        ],
    )
    def gather(table_hbm, idx_hbm, out_hbm, idx_v, rows_v, sem):
        wid = lax.axis_index("s") * _NC + lax.axis_index("c")
        base = wid * _BPW
        pltpu.sync_copy(idx_hbm.at[pl.ds(base, _BPW)], idx_v)
        pltpu.async_copy(table_hbm.at[idx_v], rows_v, sem).wait()
        pltpu.sync_copy(rows_v, out_hbm.at[pl.ds(base, _BPW)])

    return gather


def _decoder_body(e_ref, wt_ref, b_ref, out_ref, h_ref, e_out_ref):
    e = e_ref[:, :EMBED_DIM]
    norm = jnp.sqrt(jnp.sum(e * e, axis=1, keepdims=True))
    coef = jnp.minimum(CLIP_MAX_NORM / (norm + 1e-06), 1.0)
    h = e * coef

    @pl.when(pl.program_id(0) == 0)
    def _():
        h_ref[...] = h
        e_out_ref[...] = e

    acc = lax.dot_general(
        wt_ref[...], h, (((0,), (1,)), ((), ())),
        preferred_element_type=jnp.float32,
    )
    out_ref[...] = acc + jnp.transpose(b_ref[...])


@functools.lru_cache(maxsize=None)
def _make_decoder(nb: int):
    grid = (pl.cdiv(VOCAB, nb),)
    return pl.pallas_call(
        _decoder_body,
        grid=grid,
        in_specs=[
            pl.BlockSpec((BATCH, LANE), lambda n: (0, 0)),
            pl.BlockSpec((EMBED_DIM, nb), lambda n: (0, n)),
            pl.BlockSpec((1, nb), lambda n: (0, n)),
        ],
        out_specs=[
            pl.BlockSpec((nb, BATCH), lambda n: (n, 0)),
            pl.BlockSpec((BATCH, EMBED_DIM), lambda n: (0, 0)),
            pl.BlockSpec((BATCH, EMBED_DIM), lambda n: (0, 0)),
        ],
        out_shape=[
            jax.ShapeDtypeStruct((VOCAB, BATCH), jnp.float32),
            jax.ShapeDtypeStruct((BATCH, EMBED_DIM), jnp.float32),
            jax.ShapeDtypeStruct((BATCH, EMBED_DIM), jnp.float32),
        ],
        compiler_params=pltpu.CompilerParams(
            dimension_semantics=("parallel",),
        ),
    )


def kernel(x, emb_table, W_dec, b_dec):
    emb_pad = jnp.pad(emb_table, ((0, 0), (0, LANE - EMBED_DIM)))
    e128 = _make_sc_gather()(emb_pad, x.astype(jnp.int32))
    logits_t, h, e = _make_decoder(4096)(
        e128, W_dec.T, b_dec.reshape(1, VOCAB)
    )
    return (logits_t.T, h, e)
